# baseline (speedup 1.0000x reference)
import jax
import jax.numpy as jnp
from jax.experimental import pallas as pl


def _copy_body(x_ref, o_ref):
    o_ref[...] = x_ref[...]


def kernel(X):
    B, S, F = X.shape
    total = B * S * F
    assert total % 128 == 0
    rows = total // 128
    grid = 100
    assert rows % grid == 0
    blk = rows // grid

    flat = X.reshape(rows, 128)
    out = pl.pallas_call(
        _copy_body,
        grid=(grid,),
        in_specs=[pl.BlockSpec((blk, 128), lambda i: (i, 0))],
        out_specs=pl.BlockSpec((blk, 128), lambda i: (i, 0)),
        out_shape=jax.ShapeDtypeStruct((rows, 128), jnp.float32),
    )(flat)
    return out.reshape(B, S, F)
